# baseline (speedup 1.0000x reference)
import functools

import jax
import jax.numpy as jnp
from jax import lax
from jax.experimental import pallas as pl
from jax.experimental.pallas import tpu as pltpu
from jax.experimental.pallas import tpu_sc as plsc

NUM_CORES = 2
NUM_SUBCORES = 16
NW = NUM_CORES * NUM_SUBCORES

L = 128
T = 50
DIM = 64


@jax.jit
def _emb_lookup(pair, base, table2):
    mesh = plsc.VectorSubcoreMesh(core_axis_name="c", subcore_axis_name="s")

    @functools.partial(
        pl.kernel,
        mesh=mesh,
        out_type=jax.ShapeDtypeStruct((T, DIM // 8, NW, 8 * L), jnp.float32),
        scratch_types=[
            pltpu.VMEM((T, L), jnp.int32),
            pltpu.VMEM((T, L), jnp.int32),
            pltpu.VMEM((2, L, 2 * DIM), jnp.float32),
            pltpu.VMEM((2, DIM // 8, 8 * L), jnp.float32),
        ] + [pltpu.SemaphoreType.DMA] * 4,
        compiler_params=pltpu.CompilerParams(
            use_tc_tiling_on_sc=False, needs_layout_passes=False
        ),
    )
    def emb_kernel(pair_hbm, base_hbm, table_hbm, out_hbm, pid_v, base_v,
                   gbuf, obuf, gsem0, gsem1, wsem0, wsem1):
        wid = lax.axis_index("s") * NUM_CORES + lax.axis_index("c")
        gsems = (gsem0, gsem1)
        wsems = (wsem0, wsem1)
        pltpu.sync_copy(pair_hbm.at[wid], pid_v)
        pltpu.sync_copy(base_hbm.at[wid], base_v)
        pltpu.async_copy(table_hbm.at[pid_v.at[0]], gbuf.at[0], gsem0)

        lane = lax.iota(jnp.int32, 16)

        def round_body(rr, carry):
            for b in range(2):
                t = rr * 2 + b
                pltpu.make_async_copy(
                    table_hbm.at[pid_v.at[t]], gbuf.at[b], gsems[b]
                ).wait()

                @pl.when(t < T - 1)
                def _():
                    pltpu.async_copy(
                        table_hbm.at[pid_v.at[t + 1]], gbuf.at[1 - b],
                        gsems[1 - b],
                    )

                @pl.when(t >= 2)
                def _():
                    pltpu.make_async_copy(
                        obuf.at[b], out_hbm.at[t - 2].at[:, wid, :], wsems[b]
                    ).wait()

                gb = gbuf.at[b]
                ob = obuf.at[b]

                for c8 in range(DIM // 8):
                    obc = ob.at[c8]

                    def s_body(s, carry2, c8=c8, obc=obc):
                        c = c8 * 8 + s
                        for k in range(L // 16):
                            bvec = base_v[t, pl.ds(k * 16, 16)]
                            lvec = lane + (k * 16)
                            v = plsc.load_gather(gb, [lvec, bvec + c])
                            obc[pl.ds(s * L + k * 16, 16)] = v
                        return carry2

                    lax.fori_loop(0, 8, s_body, 0)

                pltpu.make_async_copy(
                    ob, out_hbm.at[t].at[:, wid, :], wsems[b]
                ).start()
            return carry

        lax.fori_loop(0, T // 2, round_body, 0)

        for b in range(2):
            t = T - 2 + b
            pltpu.make_async_copy(
                obuf.at[b], out_hbm.at[t].at[:, wid, :], wsems[b]
            ).wait()

    return emb_kernel(pair, base, table2)


def kernel(x, weight):
    xi = x.astype(jnp.int32)
    idx = xi.T.reshape(T, NW, L).transpose(1, 0, 2)
    pair = idx >> 1
    base = (idx & 1) * DIM
    table2 = weight.reshape(weight.shape[0] // 2, 2 * DIM)
    out_phys = _emb_lookup(pair, base, table2)
    out = (
        out_phys.reshape(T, DIM // 8, NW, 8, L)
        .transpose(2, 4, 0, 1, 3)
        .reshape(NW * L, T, DIM)
    )
    return out


# speedup vs baseline: 1.3736x; 1.3736x over previous
import functools

import jax
import jax.numpy as jnp
from jax import lax
from jax.experimental import pallas as pl
from jax.experimental.pallas import tpu as pltpu
from jax.experimental.pallas import tpu_sc as plsc

NUM_CORES = 2
NUM_SUBCORES = 16
NW = NUM_CORES * NUM_SUBCORES

L = 128
T = 50
DIM = 64


@jax.jit
def _emb_lookup(idx, table):
    mesh = plsc.VectorSubcoreMesh(core_axis_name="c", subcore_axis_name="s")

    @functools.partial(
        pl.kernel,
        mesh=mesh,
        out_type=jax.ShapeDtypeStruct((T, DIM // 8, NW, 8 * L), jnp.float32),
        scratch_types=[
            pltpu.VMEM((T, L), jnp.int32),
            pltpu.VMEM((2, L, DIM), jnp.float32),
            pltpu.VMEM((2, DIM // 8, 8 * L), jnp.float32),
        ] + [pltpu.SemaphoreType.DMA] * 4,
        compiler_params=pltpu.CompilerParams(
            use_tc_tiling_on_sc=False, needs_layout_passes=False
        ),
    )
    def emb_kernel(idx_hbm, table_hbm, out_hbm, idx_v, gbuf, obuf,
                   gsem0, gsem1, wsem0, wsem1):
        wid = lax.axis_index("s") * NUM_CORES + lax.axis_index("c")
        gsems = (gsem0, gsem1)
        wsems = (wsem0, wsem1)
        pltpu.sync_copy(idx_hbm.at[wid], idx_v)
        pltpu.async_copy(table_hbm.at[idx_v.at[0]], gbuf.at[0], gsem0)

        lane = lax.iota(jnp.int32, 16)
        lvecs = [lane + (k * 16) for k in range(L // 16)]

        def round_body(rr, carry):
            for b in range(2):
                t = rr * 2 + b
                pltpu.make_async_copy(
                    table_hbm.at[idx_v.at[t]], gbuf.at[b], gsems[b]
                ).wait()

                @pl.when(t < T - 1)
                def _():
                    pltpu.async_copy(
                        table_hbm.at[idx_v.at[t + 1]], gbuf.at[1 - b],
                        gsems[1 - b],
                    )

                @pl.when(t >= 2)
                def _():
                    pltpu.make_async_copy(
                        obuf.at[b], out_hbm.at[t - 2].at[:, wid, :], wsems[b]
                    ).wait()

                gb = gbuf.at[b]
                ob = obuf.at[b]

                for c8 in range(DIM // 8):
                    obc = ob.at[c8]

                    @plsc.parallel_loop(0, 8, 1, unroll=2)
                    def s_body(s, c8=c8, obc=obc):
                        c = c8 * 8 + s
                        cvec = jnp.full((16,), c, jnp.int32)
                        for k in range(L // 16):
                            v = plsc.load_gather(gb, [lvecs[k], cvec])
                            obc[pl.ds(s * L + k * 16, 16)] = v

                pltpu.make_async_copy(
                    ob, out_hbm.at[t].at[:, wid, :], wsems[b]
                ).start()
            return carry

        lax.fori_loop(0, T // 2, round_body, 0)

        for b in range(2):
            t = T - 2 + b
            pltpu.make_async_copy(
                obuf.at[b], out_hbm.at[t].at[:, wid, :], wsems[b]
            ).wait()

    return emb_kernel(idx, table)


def kernel(x, weight):
    idx = x.astype(jnp.int32).T.reshape(T, NW, L).transpose(1, 0, 2)
    out_phys = _emb_lookup(idx, weight)
    out = (
        out_phys.reshape(T, DIM // 8, NW, 8, L)
        .transpose(2, 4, 0, 1, 3)
        .reshape(NW * L, T, DIM)
    )
    return out
